# baseline (speedup 1.0000x reference)
import functools

import jax
import jax.numpy as jnp
from jax import lax
from jax.experimental import pallas as pl
from jax.experimental.pallas import tpu as pltpu
from jax.experimental.pallas import tpu_sc as plsc

S = 8


@functools.lru_cache(maxsize=None)
def _make_gather(num_samples: int, seq: int, vocab: int, embed_dim: int):
    info = plsc.get_sparse_core_info()
    nc, ns = info.num_cores, info.num_subcores
    nw = nc * ns
    assert num_samples % (nw * S) == 0
    samples_per_w = num_samples // nw
    num_chunks = samples_per_w // S
    assert num_chunks % 2 == 0

    mesh = plsc.VectorSubcoreMesh(core_axis_name="c", subcore_axis_name="s")

    @functools.partial(
        pl.kernel,
        mesh=mesh,
        out_type=jax.ShapeDtypeStruct((num_samples, seq, embed_dim),
                                      jnp.float32),
        compiler_params=pltpu.CompilerParams(use_tc_tiling_on_sc=False),
        scratch_types=(
            [pltpu.VMEM((seq,), jnp.int32) for _ in range(2 * S)]
            + [pltpu.VMEM((S, seq, embed_dim), jnp.float32) for _ in range(2)]
            + [pltpu.SemaphoreType.DMA for _ in range(6)]
        ),
    )
    def gather_kernel(table_hbm, idx_hbm, out_hbm, *scratch):
        idx_vs = (scratch[:S], scratch[S:2 * S])
        rows_vs = scratch[2 * S:2 * S + 2]
        gsem = scratch[2 * S + 2:2 * S + 4]
        isem = scratch[2 * S + 4:2 * S + 6]
        osem = scratch[2 * S + 6:2 * S + 8]

        wid = lax.axis_index("s") * nc + lax.axis_index("c")
        base = wid * samples_per_w

        def fire_idx(b, c):
            for j in range(S):
                pltpu.async_copy(idx_hbm.at[base + c * S + j],
                                 idx_vs[b][j], isem[b])

        def drain_idx(b, c):
            for j in range(S):
                pltpu.make_async_copy(idx_hbm.at[base + c * S + j],
                                      idx_vs[b][j], isem[b]).wait()

        def fire_gathers(b):
            for j in range(S):
                pltpu.async_copy(table_hbm.at[idx_vs[b][j]],
                                 rows_vs[b].at[j], gsem[b])

        def drain_gathers(b):
            for j in range(S):
                pltpu.make_async_copy(table_hbm.at[idx_vs[b][j]],
                                      rows_vs[b].at[j], gsem[b]).wait()

        def fire_write(b, c):
            pltpu.async_copy(rows_vs[b],
                             out_hbm.at[pl.ds(base + c * S, S)], osem[b])

        def drain_write(b, c):
            pltpu.make_async_copy(rows_vs[b],
                                  out_hbm.at[pl.ds(base + c * S, S)],
                                  osem[b]).wait()

        for j in range(S):
            pltpu.sync_copy(idx_hbm.at[base + j], idx_vs[0][j])
        fire_gathers(0)
        fire_idx(1, 1)

        def pair_body(s, carry):
            for half in range(2):
                c = 2 * s + half
                b = half

                @pl.when(c + 1 < num_chunks)
                def _():
                    drain_idx(1 - b, c + 1)

                @pl.when(c > 0)
                def _():
                    drain_write(1 - b, c - 1)

                @pl.when(c + 1 < num_chunks)
                def _():
                    fire_gathers(1 - b)

                drain_gathers(b)
                fire_write(b, c)

                @pl.when(c + 2 < num_chunks)
                def _():
                    fire_idx(b, c + 2)

            return carry

        lax.fori_loop(0, num_chunks // 2, pair_body, 0)
        drain_write(1, num_chunks - 1)

    return gather_kernel


def kernel(x, table):
    vocab, embed_dim = table.shape
    num_samples, seq = x.shape
    fn = _make_gather(num_samples, seq, vocab, embed_dim)
    return fn(table, x.astype(jnp.int32))
